# baseline (speedup 1.0000x reference)
import functools

import jax
import jax.numpy as jnp
from jax.experimental import pallas as pl
from jax.experimental.pallas import tpu as pltpu

_ALPHA = 32.0


def _round_up(x, m):
    return ((x + m - 1) // m) * m


def _weff_kernel(wt_ref, a_ref, bmat_ref, weff_ref, *, scaling):
    lora = jnp.dot(a_ref[...], bmat_ref[...],
                   preferred_element_type=jnp.float32)
    weff_ref[...] = (wt_ref[...] + scaling * lora).astype(jnp.bfloat16)


def _main_kernel(x_ref, weff_ref, b_ref, o_ref, *, tn):
    j = pl.program_id(1)
    xb = x_ref[...].astype(jnp.bfloat16)
    acc = jnp.dot(xb, weff_ref[...], preferred_element_type=jnp.float32)
    acc += b_ref[:, pl.ds(j * tn, tn)]
    o_ref[...] = acc.astype(o_ref.dtype)


def kernel(x, wt, b, a, bmat):
    orig_shape = x.shape
    in_dim = orig_shape[-1]
    out_dim = wt.shape[1]
    rank = a.shape[1]
    scaling = _ALPHA / float(rank)

    x2d = x.reshape(-1, in_dim)
    M = x2d.shape[0]

    tm = min(1024, _round_up(M, 8))
    tn = min(1024, _round_up(out_dim, 128))
    tn_w = min(512, _round_up(out_dim, 128))

    M_pad = _round_up(M, tm)
    K_pad = _round_up(in_dim, 128)
    N_pad = _round_up(out_dim, max(tn, tn_w))
    r_pad = _round_up(rank, 8)

    if M_pad != M or K_pad != in_dim:
        x2d = jnp.pad(x2d, ((0, M_pad - M), (0, K_pad - in_dim)))
    if K_pad != in_dim or N_pad != out_dim:
        wt = jnp.pad(wt, ((0, K_pad - in_dim), (0, N_pad - out_dim)))
    if K_pad != in_dim or r_pad != rank:
        a = jnp.pad(a, ((0, K_pad - in_dim), (0, r_pad - rank)))
    if r_pad != rank or N_pad != out_dim:
        bmat = jnp.pad(bmat, ((0, r_pad - rank), (0, N_pad - out_dim)))
    if N_pad != out_dim:
        b = jnp.pad(b, ((0, N_pad - out_dim),))
    b2d = b.reshape(1, N_pad)

    vmem_limit = 100 * 1024 * 1024

    weff = pl.pallas_call(
        functools.partial(_weff_kernel, scaling=scaling),
        out_shape=jax.ShapeDtypeStruct((K_pad, N_pad), jnp.bfloat16),
        grid=(N_pad // tn_w,),
        in_specs=[
            pl.BlockSpec((K_pad, tn_w), lambda j: (0, j)),
            pl.BlockSpec((K_pad, r_pad), lambda j: (0, 0)),
            pl.BlockSpec((r_pad, tn_w), lambda j: (0, j)),
        ],
        out_specs=pl.BlockSpec((K_pad, tn_w), lambda j: (0, j)),
        compiler_params=pltpu.CompilerParams(
            dimension_semantics=("arbitrary",),
            vmem_limit_bytes=vmem_limit),
    )(wt, a, bmat)

    out2d = pl.pallas_call(
        functools.partial(_main_kernel, tn=tn),
        out_shape=jax.ShapeDtypeStruct((M_pad, N_pad), x.dtype),
        grid=(M_pad // tm, N_pad // tn),
        in_specs=[
            pl.BlockSpec((tm, K_pad), lambda i, j: (i, 0)),
            pl.BlockSpec((K_pad, tn), lambda i, j: (0, j)),
            pl.BlockSpec((1, N_pad), lambda i, j: (0, 0)),
        ],
        out_specs=pl.BlockSpec((tm, tn), lambda i, j: (i, j)),
        compiler_params=pltpu.CompilerParams(
            dimension_semantics=("arbitrary", "arbitrary"),
            vmem_limit_bytes=vmem_limit),
    )(x2d, weff, b2d)

    out2d = out2d[:M, :out_dim]
    return out2d.reshape(*orig_shape[:-1], out_dim)


# speedup vs baseline: 1.0037x; 1.0037x over previous
import functools

import jax
import jax.numpy as jnp
from jax.experimental import pallas as pl
from jax.experimental.pallas import tpu as pltpu

_ALPHA = 32.0


def _round_up(x, m):
    return ((x + m - 1) // m) * m


def _weff_kernel(wt_ref, a_ref, bmat_ref, weff_ref, *, scaling):
    lora = jnp.dot(a_ref[...], bmat_ref[...],
                   preferred_element_type=jnp.float32)
    weff_ref[...] = (wt_ref[...] + scaling * lora).astype(jnp.bfloat16)


def _main_kernel(x_ref, weff_ref, b_ref, o_ref, *, tn):
    j = pl.program_id(1)
    xb = x_ref[...].astype(jnp.bfloat16)
    acc = jnp.dot(xb, weff_ref[...], preferred_element_type=jnp.float32)
    acc += b_ref[:, pl.ds(j * tn, tn)]
    o_ref[...] = acc.astype(o_ref.dtype)


def kernel(x, wt, b, a, bmat):
    orig_shape = x.shape
    in_dim = orig_shape[-1]
    out_dim = wt.shape[1]
    rank = a.shape[1]
    scaling = _ALPHA / float(rank)

    x2d = x.reshape(-1, in_dim)
    M = x2d.shape[0]

    tm = min(1024, _round_up(M, 8))
    tn = min(1024, _round_up(out_dim, 128))
    tn_w = min(512, _round_up(out_dim, 128))

    M_pad = _round_up(M, tm)
    K_pad = _round_up(in_dim, 128)
    N_pad = _round_up(out_dim, max(tn, tn_w))
    if N_pad % tn_w:
        tn_w = tn
    r_pad = _round_up(rank, 8)

    if M_pad != M or K_pad != in_dim:
        x2d = jnp.pad(x2d, ((0, M_pad - M), (0, K_pad - in_dim)))
    if K_pad != in_dim or N_pad != out_dim:
        wt = jnp.pad(wt, ((0, K_pad - in_dim), (0, N_pad - out_dim)))
    if K_pad != in_dim or r_pad != rank:
        a = jnp.pad(a, ((0, K_pad - in_dim), (0, r_pad - rank)))
    if r_pad != rank or N_pad != out_dim:
        bmat = jnp.pad(bmat, ((0, r_pad - rank), (0, N_pad - out_dim)))
    if N_pad != out_dim:
        b = jnp.pad(b, ((0, N_pad - out_dim),))
    b2d = b.reshape(1, N_pad)

    vmem_limit = 100 * 1024 * 1024

    weff = pl.pallas_call(
        functools.partial(_weff_kernel, scaling=scaling),
        out_shape=jax.ShapeDtypeStruct((K_pad, N_pad), jnp.bfloat16),
        grid=(N_pad // tn_w,),
        in_specs=[
            pl.BlockSpec((K_pad, tn_w), lambda j: (0, j)),
            pl.BlockSpec((K_pad, r_pad), lambda j: (0, 0)),
            pl.BlockSpec((r_pad, tn_w), lambda j: (0, j)),
        ],
        out_specs=pl.BlockSpec((K_pad, tn_w), lambda j: (0, j)),
        compiler_params=pltpu.CompilerParams(
            dimension_semantics=("arbitrary",),
            vmem_limit_bytes=vmem_limit),
    )(wt, a, bmat)

    out2d = pl.pallas_call(
        functools.partial(_main_kernel, tn=tn),
        out_shape=jax.ShapeDtypeStruct((M_pad, N_pad), x.dtype),
        grid=(M_pad // tm, N_pad // tn),
        in_specs=[
            pl.BlockSpec((tm, K_pad), lambda i, j: (i, 0)),
            pl.BlockSpec((K_pad, tn), lambda i, j: (0, j)),
            pl.BlockSpec((1, N_pad), lambda i, j: (0, 0)),
        ],
        out_specs=pl.BlockSpec((tm, tn), lambda i, j: (i, j)),
        compiler_params=pltpu.CompilerParams(
            dimension_semantics=("arbitrary", "arbitrary"),
            vmem_limit_bytes=vmem_limit),
    )(x2d, weff, b2d)

    out2d = out2d[:M, :out_dim]
    return out2d.reshape(*orig_shape[:-1], out_dim)
